# baseline (speedup 1.0000x reference)
import functools

import jax
import jax.numpy as jnp
from jax import lax
from jax.experimental import pallas as pl
from jax.experimental.pallas import tpu as pltpu
from jax.experimental.pallas import tpu_sc as plsc

_NC = 2
_NS = 16
_CH = 128
_NB = 3


def _pick_rows(n):
    for r in (1000, 500, 250, 200, 125, 100, 50, 40, 25, 20, 10, 8):
        if n % r == 0:
            return r
    return n


def _ln_split_body(x_ref, g_ref, b_ref, o_ref):
    x = x_ref[...]
    mu = jnp.mean(x, axis=-1, keepdims=True)
    var = jnp.mean((x - mu) ** 2, axis=-1, keepdims=True)
    h = (x - mu) * lax.rsqrt(var + 1e-5) * g_ref[...] + b_ref[...]
    d2 = x.shape[-1] // 2
    o_ref[0] = h[:, :d2]
    o_ref[1] = h[:, d2:]


def _layernorm_split(x, g, b):
    n, d = x.shape
    r = _pick_rows(n)
    return pl.pallas_call(
        _ln_split_body,
        grid=(n // r,),
        in_specs=[
            pl.BlockSpec((r, d), lambda i: (i, 0)),
            pl.BlockSpec((1, d), lambda i: (0, 0)),
            pl.BlockSpec((1, d), lambda i: (0, 0)),
        ],
        out_specs=pl.BlockSpec((2, r, d // 2), lambda i: (0, i, 0)),
        out_shape=jax.ShapeDtypeStruct((2, n, d // 2), jnp.float32),
    )(x, g.reshape(1, d), b.reshape(1, d))


def _sc_aggregate(hsplit, ei3, n, n_pad):
    d2 = hsplit.shape[2]
    erows = ei3.shape[1]
    rpw = -(-erows // _NS)
    last = erows - rpw * (_NS - 1)
    steps = ((rpw + _NB - 1) // _NB) * _NB
    rps = n_pad // _NS
    mesh = plsc.VectorSubcoreMesh(core_axis_name="c", subcore_axis_name="s")

    @functools.partial(
        pl.kernel,
        out_type=[
            jax.ShapeDtypeStruct((n_pad, _NC * d2), jnp.float32),
            jax.ShapeDtypeStruct((n_pad, 32), jnp.float32),
        ],
        mesh=mesh,
        scratch_types=[
            pltpu.VMEM((steps, _CH), jnp.int32),
            pltpu.VMEM((steps, _CH), jnp.int32),
            pltpu.VMEM((_NB, _CH, d2), jnp.float32),
            pltpu.VMEM((_CH, 16), jnp.float32),
            pltpu.VMEM((32, d2), jnp.float32),
            pltpu.VMEM((32, 16), jnp.float32),
            pltpu.VMEM_SHARED((n_pad, d2), jnp.float32),
            pltpu.VMEM_SHARED((n_pad, 16), jnp.float32),
        ] + [pltpu.SemaphoreType.DMA] * (2 * _NB + 1),
        compiler_params=pltpu.CompilerParams(use_tc_tiling_on_sc=False),
    )
    def agg(h_hbm, ei_hbm, sums_out, cnt_out,
            sidx, didx, rows, ones_v, z_d, z_16, acc, accc, *sems):
        gsem = sems[:_NB]
        ssem = sems[_NB:2 * _NB]
        osem = sems[2 * _NB]
        cid = lax.axis_index("c")
        sid = lax.axis_index("s")

        row0 = sid * rpw
        nrows = jnp.where(sid == _NS - 1, last, rpw)

        @pl.when(sid == _NS - 1)
        def _():
            pltpu.async_copy(ei_hbm.at[0, pl.ds(row0, last)],
                             sidx.at[pl.ds(0, last)], gsem[0])
            pltpu.async_copy(ei_hbm.at[1, pl.ds(row0, last)],
                             didx.at[pl.ds(0, last)], gsem[1])

        @pl.when(sid != _NS - 1)
        def _():
            pltpu.async_copy(ei_hbm.at[0, pl.ds(row0, rpw)],
                             sidx.at[pl.ds(0, rpw)], gsem[0])
            pltpu.async_copy(ei_hbm.at[1, pl.ds(row0, rpw)],
                             didx.at[pl.ds(0, rpw)], gsem[1])

        @pl.loop(0, 32)
        def _(r):
            @pl.loop(0, d2, step=16)
            def _(c0):
                z_d[r, pl.ds(c0, 16)] = jnp.zeros((16,), jnp.float32)
            z_16[r, :] = jnp.zeros((16,), jnp.float32)

        @pl.loop(0, _CH)
        def _(r):
            ones_v[r, :] = jnp.ones((16,), jnp.float32)

        base = sid * rps

        @pl.loop(0, rps, step=32)
        def _(r0):
            pltpu.async_copy(z_d, acc.at[pl.ds(base + r0, 32)], ssem[0])
            pltpu.async_copy(z_16, accc.at[pl.ds(base + r0, 32)], ssem[1])

        @pl.when(sid == _NS - 1)
        def _():
            pltpu.make_async_copy(ei_hbm.at[0, pl.ds(0, last)],
                                  sidx.at[pl.ds(0, last)], gsem[0]).wait()
            pltpu.make_async_copy(ei_hbm.at[1, pl.ds(0, last)],
                                  didx.at[pl.ds(0, last)], gsem[1]).wait()

        @pl.when(sid != _NS - 1)
        def _():
            pltpu.make_async_copy(ei_hbm.at[0, pl.ds(0, rpw)],
                                  sidx.at[pl.ds(0, rpw)], gsem[0]).wait()
            pltpu.make_async_copy(ei_hbm.at[1, pl.ds(0, rpw)],
                                  didx.at[pl.ds(0, rpw)], gsem[1]).wait()

        @pl.loop(0, steps - last)
        def _(i):
            r = nrows + i

            @pl.when(r < steps)
            def _():
                @pl.loop(0, _CH, step=16)
                def _(c0):
                    sidx[r, pl.ds(c0, 16)] = jnp.zeros((16,), jnp.int32)
                    didx[r, pl.ds(c0, 16)] = jnp.full((16,), n, jnp.int32)

        h_pl = h_hbm.at[cid]

        for b in range(_NB):
            pltpu.async_copy(h_pl.at[sidx.at[b]], rows.at[b], gsem[b])

        @pl.loop(0, rps // 32)
        def _(i):
            pltpu.make_async_copy(z_d, acc.at[pl.ds(base, 32)],
                                  ssem[0]).wait()
            pltpu.make_async_copy(z_16, accc.at[pl.ds(base, 32)],
                                  ssem[1]).wait()

        plsc.subcore_barrier()

        half = steps // 2

        @pl.loop(0, steps, step=_NB)
        def _(j):
            for b in range(_NB):
                k = j + b
                pltpu.make_async_copy(
                    h_pl.at[sidx.at[k]], rows.at[b], gsem[b]).wait()
                pltpu.async_copy(rows.at[b], acc.at[didx.at[k]], ssem[b],
                                 add=True)

                @pl.when((k < half) == (cid == 0))
                def _():
                    pltpu.async_copy(ones_v, accc.at[didx.at[k]], osem,
                                     add=True)

                pb = (b - 1) % _NB
                kp = k - 1

                @pl.when((kp >= 0) & (kp + _NB < steps))
                def _():
                    pltpu.make_async_copy(
                        rows.at[pb], acc.at[didx.at[kp]], ssem[pb]).wait()
                    pltpu.async_copy(
                        h_pl.at[sidx.at[kp + _NB]], rows.at[pb], gsem[pb])

        for b in range(_NB):
            pltpu.make_async_copy(rows.at[b], acc.at[didx.at[0]],
                                  ssem[b]).wait()

        n_ones = jnp.where(cid == 0, half, steps - half)

        @pl.loop(0, n_ones)
        def _(i):
            pltpu.make_async_copy(ones_v, accc.at[didx.at[0]], osem).wait()

        plsc.subcore_barrier()

        pltpu.sync_copy(acc.at[pl.ds(base, rps)],
                        sums_out.at[pl.ds(base, rps),
                                    pl.ds(cid * d2, d2)])
        pltpu.sync_copy(accc.at[pl.ds(base, rps)],
                        cnt_out.at[pl.ds(base, rps),
                                   pl.ds(cid * 16, 16)])

    return agg(hsplit, ei3)


def _dot(a, b):
    return jnp.dot(a, b, preferred_element_type=jnp.float32)


def _self_body(h_ref, ws_ref, bs_ref, o_ref):
    h = jnp.concatenate([h_ref[0], h_ref[1]], axis=-1)
    nh = ws_ref.shape[0]
    d = h.shape[-1]
    for i in range(nh):
        o_ref[:, i * d:(i + 1) * d] = _dot(h, ws_ref[i]) + bs_ref[i]


def _self_cat(hsplit, w_self, b_sage):
    _, n, d2 = hsplit.shape
    nh, d, _ = w_self.shape
    r = _pick_rows(n)
    full = lambda shape: pl.BlockSpec(shape, lambda i: tuple(0 for _ in shape))
    return pl.pallas_call(
        _self_body,
        grid=(n // r,),
        in_specs=[
            pl.BlockSpec((_NC, r, d2), lambda i: (0, i, 0)),
            full((nh, d, d)), full((nh, d)),
        ],
        out_specs=pl.BlockSpec((r, nh * d), lambda i: (i, 0)),
        out_shape=jax.ShapeDtypeStruct((n, nh * d), jnp.float32),
    )(hsplit, w_self, b_sage)


def _tail_body(x_ref, selfcat_ref, sums_ref, cnt_ref, wn_ref,
               wfc_ref, bfc_ref, w1_ref, b1_ref, w2_ref, b2_ref,
               g2_ref, be2_ref, o_ref):
    x = x_ref[...]
    cnt = cnt_ref[:, :1] + cnt_ref[:, 16:17]
    h_neigh = sums_ref[...] / jnp.maximum(cnt, 1.0)
    nh = wn_ref.shape[0]
    d = x.shape[-1]
    gnn = bfc_ref[...]
    for i in range(nh):
        ci = selfcat_ref[:, i * d:(i + 1) * d] + _dot(h_neigh, wn_ref[i])
        gnn = gnn + _dot(jnp.maximum(ci, 0.0),
                         wfc_ref[pl.ds(i * d, d), :])
    x1 = x + gnn
    mu = jnp.mean(x1, axis=-1, keepdims=True)
    var = jnp.mean((x1 - mu) ** 2, axis=-1, keepdims=True)
    h2 = (x1 - mu) * lax.rsqrt(var + 1e-5) * g2_ref[...] + be2_ref[...]
    ffn = _dot(jnp.maximum(_dot(h2, w1_ref[...]) + b1_ref[...], 0.0),
               w2_ref[...]) + b2_ref[...]
    o_ref[...] = x1 + ffn


def _tail(x, selfcat, sums2, cnt2, w_neigh, wfc, bfc,
          w1, b1, w2, b2, g2, be2):
    n, d = x.shape
    nh = w_neigh.shape[0]
    hd = nh * d
    r = _pick_rows(n)
    full = lambda shape: pl.BlockSpec(shape, lambda i: tuple(0 for _ in shape))
    return pl.pallas_call(
        _tail_body,
        grid=(n // r,),
        in_specs=[
            pl.BlockSpec((r, d), lambda i: (i, 0)),
            pl.BlockSpec((r, hd), lambda i: (i, 0)),
            pl.BlockSpec((r, d), lambda i: (i, 0)),
            pl.BlockSpec((r, 32), lambda i: (i, 0)),
            full((nh, d, d)),
            full((hd, d)), full((1, d)),
            full((d, d)), full((1, d)),
            full((d, d)), full((1, d)),
            full((1, d)), full((1, d)),
        ],
        out_specs=pl.BlockSpec((r, d), lambda i: (i, 0)),
        out_shape=jax.ShapeDtypeStruct((n, d), jnp.float32),
    )(x, selfcat, sums2, cnt2, w_neigh, wfc, bfc,
      w1, b1, w2, b2, g2, be2)


def kernel(x, edge_index, W_self, W_neigh, b_sage, W_fc, b_fc,
           W1, b1, W2, b2, g1, be1, g2, be2):
    n, d = x.shape
    e = edge_index.shape[1]
    n_pad = -(-(n + 1) // (_NS * 32)) * (_NS * 32)

    ei3 = edge_index.reshape(2, e // _CH, _CH)

    hsplit = _layernorm_split(x, g1, be1)
    sums2, cnt2 = _sc_aggregate(hsplit, ei3, n, n_pad)
    selfcat = _self_cat(hsplit, W_self, b_sage)

    return _tail(x, selfcat, sums2, cnt2, W_neigh,
                 W_fc, b_fc.reshape(1, d), W1, b1.reshape(1, d),
                 W2, b2.reshape(1, d), g2.reshape(1, d), be2.reshape(1, d))


# speedup vs baseline: 1.0306x; 1.0306x over previous
import functools

import jax
import jax.numpy as jnp
from jax import lax
from jax.experimental import pallas as pl
from jax.experimental.pallas import tpu as pltpu
from jax.experimental.pallas import tpu_sc as plsc

_NC = 2
_NS = 16
_CH = 128
_NB = 3


def _pick_rows(n):
    for r in (1000, 500, 250, 200, 125, 100, 50, 40, 25, 20, 10, 8):
        if n % r == 0:
            return r
    return n


def _ln_split_body(x_ref, g_ref, b_ref, o_ref):
    x = x_ref[...]
    mu = jnp.mean(x, axis=-1, keepdims=True)
    var = jnp.mean((x - mu) ** 2, axis=-1, keepdims=True)
    h = (x - mu) * lax.rsqrt(var + 1e-5) * g_ref[...] + b_ref[...]
    d2 = x.shape[-1] // 2
    o_ref[0] = h[:, :d2]
    o_ref[1] = h[:, d2:]


def _layernorm_split(x, g, b):
    n, d = x.shape
    r = _pick_rows(n)
    return pl.pallas_call(
        _ln_split_body,
        grid=(n // r,),
        in_specs=[
            pl.BlockSpec((r, d), lambda i: (i, 0)),
            pl.BlockSpec((1, d), lambda i: (0, 0)),
            pl.BlockSpec((1, d), lambda i: (0, 0)),
        ],
        out_specs=pl.BlockSpec((2, r, d // 2), lambda i: (0, i, 0)),
        out_shape=jax.ShapeDtypeStruct((2, n, d // 2), jnp.float32),
    )(x, g.reshape(1, d), b.reshape(1, d))


def _sc_aggregate(hsplit, ei3, n, n_pad):
    d2 = hsplit.shape[2]
    erows = ei3.shape[1]
    rpw = -(-erows // _NS)
    last = erows - rpw * (_NS - 1)
    steps = ((rpw + _NB - 1) // _NB) * _NB
    rps = n_pad // _NS
    mesh = plsc.VectorSubcoreMesh(core_axis_name="c", subcore_axis_name="s")

    @functools.partial(
        pl.kernel,
        out_type=[
            jax.ShapeDtypeStruct((n_pad, _NC * d2), jnp.float32),
            jax.ShapeDtypeStruct((_NC, n_pad // 128, 128), jnp.float32),
        ],
        mesh=mesh,
        scratch_types=[
            pltpu.VMEM((steps, _CH), jnp.int32),
            pltpu.VMEM((steps, _CH), jnp.int32),
            pltpu.VMEM((_NB, _CH, d2), jnp.float32),
            pltpu.VMEM((32, d2), jnp.float32),
            pltpu.VMEM((n_pad // 128, 128), jnp.float32),
            pltpu.VMEM((80,), jnp.int32),
            pltpu.VMEM_SHARED((n_pad, d2), jnp.float32),
            pltpu.VMEM_SHARED((n_pad // 128, 128), jnp.float32),
        ] + [pltpu.SemaphoreType.DMA] * (2 * _NB),
        compiler_params=pltpu.CompilerParams(use_tc_tiling_on_sc=False,
                                             needs_layout_passes=False),
    )
    def agg(h_hbm, ei_hbm, sums_out, cnt_out,
            sidx, didx, rows, z_d, cnt_t, idv, acc, accc2, *sems):
        gsem = sems[:_NB]
        ssem = sems[_NB:2 * _NB]
        cid = lax.axis_index("c")
        sid = lax.axis_index("s")

        row0 = sid * rpw
        nrows = jnp.where(sid == _NS - 1, last, rpw)

        @pl.when(sid == _NS - 1)
        def _():
            pltpu.async_copy(ei_hbm.at[0, pl.ds(row0, last)],
                             sidx.at[pl.ds(0, last)], gsem[0])
            pltpu.async_copy(ei_hbm.at[1, pl.ds(row0, last)],
                             didx.at[pl.ds(0, last)], gsem[1])

        @pl.when(sid != _NS - 1)
        def _():
            pltpu.async_copy(ei_hbm.at[0, pl.ds(row0, rpw)],
                             sidx.at[pl.ds(0, rpw)], gsem[0])
            pltpu.async_copy(ei_hbm.at[1, pl.ds(row0, rpw)],
                             didx.at[pl.ds(0, rpw)], gsem[1])

        @pl.loop(0, 32)
        def _(r):
            @pl.loop(0, d2, step=16)
            def _(c0):
                z_d[r, pl.ds(c0, 16)] = jnp.zeros((16,), jnp.float32)

        @pl.loop(0, n_pad // 128)
        def _(r):
            @pl.loop(0, 128, step=16)
            def _(c0):
                cnt_t[r, pl.ds(c0, 16)] = jnp.zeros((16,), jnp.float32)

        for i in range(5):
            idv[pl.ds(i * 16, 16)] = lax.iota(jnp.int32, 16) + (i * 16)

        base = sid * rps

        @pl.loop(0, rps, step=32)
        def _(r0):
            pltpu.async_copy(z_d, acc.at[pl.ds(base + r0, 32)], ssem[0])

        r5 = n_pad // 128 // _NS
        pltpu.async_copy(cnt_t.at[pl.ds(sid * r5, r5)],
                         accc2.at[pl.ds(sid * r5, r5)], ssem[1])

        @pl.when(sid == _NS - 1)
        def _():
            pltpu.make_async_copy(ei_hbm.at[0, pl.ds(0, last)],
                                  sidx.at[pl.ds(0, last)], gsem[0]).wait()
            pltpu.make_async_copy(ei_hbm.at[1, pl.ds(0, last)],
                                  didx.at[pl.ds(0, last)], gsem[1]).wait()

        @pl.when(sid != _NS - 1)
        def _():
            pltpu.make_async_copy(ei_hbm.at[0, pl.ds(0, rpw)],
                                  sidx.at[pl.ds(0, rpw)], gsem[0]).wait()
            pltpu.make_async_copy(ei_hbm.at[1, pl.ds(0, rpw)],
                                  didx.at[pl.ds(0, rpw)], gsem[1]).wait()

        @pl.loop(0, steps - last)
        def _(i):
            r = nrows + i

            @pl.when(r < steps)
            def _():
                @pl.loop(0, _CH, step=16)
                def _(c0):
                    sidx[r, pl.ds(c0, 16)] = jnp.zeros((16,), jnp.int32)
                    didx[r, pl.ds(c0, 16)] = jnp.full((16,), n, jnp.int32)

        h_pl = h_hbm.at[cid]

        for b in range(_NB):
            pltpu.async_copy(h_pl.at[sidx.at[b]], rows.at[b], gsem[b])

        @pl.loop(0, rps // 32)
        def _(i):
            pltpu.make_async_copy(z_d, acc.at[pl.ds(base, 32)],
                                  ssem[0]).wait()

        r5 = n_pad // 128 // _NS
        pltpu.make_async_copy(cnt_t.at[pl.ds(0, r5)],
                              accc2.at[pl.ds(0, r5)], ssem[1]).wait()

        plsc.subcore_barrier()

        half = steps // 2

        @pl.loop(0, steps, step=_NB)
        def _(j):
            for b in range(_NB):
                k = j + b
                pltpu.make_async_copy(
                    h_pl.at[sidx.at[k]], rows.at[b], gsem[b]).wait()
                pltpu.async_copy(rows.at[b], acc.at[didx.at[k]], ssem[b],
                                 add=True)

                @pl.when((k < half) == (cid == 0))
                def _():
                    for c0 in range(0, _CH, 16):
                        iv = didx[k, pl.ds(c0, 16)]
                        plsc.addupdate_scatter(
                            cnt_t,
                            [lax.shift_right_logical(iv, 7),
                             lax.bitwise_and(iv, 127)],
                            jnp.ones((16,), jnp.float32))

                pb = (b - 1) % _NB
                kp = k - 1

                @pl.when((kp >= 0) & (kp + _NB < steps))
                def _():
                    pltpu.make_async_copy(
                        rows.at[pb], acc.at[didx.at[kp]], ssem[pb]).wait()
                    pltpu.async_copy(
                        h_pl.at[sidx.at[kp + _NB]], rows.at[pb], gsem[pb])

        for b in range(_NB):
            pltpu.make_async_copy(rows.at[b], acc.at[didx.at[0]],
                                  ssem[b]).wait()

        pltpu.sync_copy(cnt_t, accc2.at[idv], add=True)

        plsc.subcore_barrier()

        pltpu.sync_copy(acc.at[pl.ds(base, rps)],
                        sums_out.at[pl.ds(base, rps),
                                    pl.ds(cid * d2, d2)])
        pltpu.sync_copy(accc2.at[pl.ds(sid * r5, r5)],
                        cnt_out.at[cid, pl.ds(sid * r5, r5)])

    return agg(hsplit, ei3)


def _dot(a, b):
    return jnp.dot(a, b, preferred_element_type=jnp.float32)


def _self_body(h_ref, ws_ref, bs_ref, o_ref):
    h = jnp.concatenate([h_ref[0], h_ref[1]], axis=-1)
    nh = ws_ref.shape[0]
    d = h.shape[-1]
    for i in range(nh):
        o_ref[:, i * d:(i + 1) * d] = _dot(h, ws_ref[i]) + bs_ref[i]


def _self_cat(hsplit, w_self, b_sage):
    _, n, d2 = hsplit.shape
    nh, d, _ = w_self.shape
    r = _pick_rows(n)
    full = lambda shape: pl.BlockSpec(shape, lambda i: tuple(0 for _ in shape))
    return pl.pallas_call(
        _self_body,
        grid=(n // r,),
        in_specs=[
            pl.BlockSpec((_NC, r, d2), lambda i: (0, i, 0)),
            full((nh, d, d)), full((nh, d)),
        ],
        out_specs=pl.BlockSpec((r, nh * d), lambda i: (i, 0)),
        out_shape=jax.ShapeDtypeStruct((n, nh * d), jnp.float32),
    )(hsplit, w_self, b_sage)


def _tail_body(x_ref, selfcat_ref, sums_ref, cnt_ref, wn_ref,
               wfc_ref, bfc_ref, w1_ref, b1_ref, w2_ref, b2_ref,
               g2_ref, be2_ref, o_ref):
    x = x_ref[...]
    cnt = cnt_ref[...]
    h_neigh = sums_ref[...] / jnp.maximum(cnt, 1.0)
    nh = wn_ref.shape[0]
    d = x.shape[-1]
    gnn = bfc_ref[...]
    for i in range(nh):
        ci = selfcat_ref[:, i * d:(i + 1) * d] + _dot(h_neigh, wn_ref[i])
        gnn = gnn + _dot(jnp.maximum(ci, 0.0),
                         wfc_ref[pl.ds(i * d, d), :])
    x1 = x + gnn
    mu = jnp.mean(x1, axis=-1, keepdims=True)
    var = jnp.mean((x1 - mu) ** 2, axis=-1, keepdims=True)
    h2 = (x1 - mu) * lax.rsqrt(var + 1e-5) * g2_ref[...] + be2_ref[...]
    ffn = _dot(jnp.maximum(_dot(h2, w1_ref[...]) + b1_ref[...], 0.0),
               w2_ref[...]) + b2_ref[...]
    o_ref[...] = x1 + ffn


def _tail(x, selfcat, sums2, cnt2, w_neigh, wfc, bfc,
          w1, b1, w2, b2, g2, be2):
    n, d = x.shape
    nh = w_neigh.shape[0]
    hd = nh * d
    r = 1280
    full = lambda shape: pl.BlockSpec(shape, lambda i: tuple(0 for _ in shape))
    return pl.pallas_call(
        _tail_body,
        grid=(-(-n // r),),
        in_specs=[
            pl.BlockSpec((r, d), lambda i: (i, 0)),
            pl.BlockSpec((r, hd), lambda i: (i, 0)),
            pl.BlockSpec((r, d), lambda i: (i, 0)),
            pl.BlockSpec((r, 1), lambda i: (i, 0)),
            full((nh, d, d)),
            full((hd, d)), full((1, d)),
            full((d, d)), full((1, d)),
            full((d, d)), full((1, d)),
            full((1, d)), full((1, d)),
        ],
        out_specs=pl.BlockSpec((r, d), lambda i: (i, 0)),
        out_shape=jax.ShapeDtypeStruct((n, d), jnp.float32),
    )(x, selfcat, sums2, cnt2, w_neigh, wfc, bfc,
      w1, b1, w2, b2, g2, be2)


def kernel(x, edge_index, W_self, W_neigh, b_sage, W_fc, b_fc,
           W1, b1, W2, b2, g1, be1, g2, be2):
    n, d = x.shape
    e = edge_index.shape[1]
    n_pad = -(-(n + 1) // (_NS * 32)) * (_NS * 32)

    ei3 = edge_index.reshape(2, e // _CH, _CH)

    hsplit = _layernorm_split(x, g1, be1)
    sums2, cnt2 = _sc_aggregate(hsplit, ei3, n, n_pad)
    cntcol = (cnt2[0] + cnt2[1]).reshape(n_pad, 1)
    selfcat = _self_cat(hsplit, W_self, b_sage)

    return _tail(x, selfcat, sums2, cntcol, W_neigh,
                 W_fc, b_fc.reshape(1, d), W1, b1.reshape(1, d),
                 W2, b2.reshape(1, d), g2.reshape(1, d), be2.reshape(1, d))
